# baseline (speedup 1.0000x reference)
import functools

import jax
import jax.numpy as jnp
from jax import lax
from jax.experimental import pallas as pl
from jax.experimental.pallas import tpu as pltpu
from jax.experimental.pallas import tpu_sc as plsc

_NBUF = 4


def _sc_gather(tok_table, flat_idx, chunk):
    n, = flat_idx.shape
    d = tok_table.shape[1]
    info = plsc.get_sparse_core_info()
    nc, ns = info.num_cores, info.num_subcores
    nw = nc * ns
    per_w = n // nw
    n_chunks = per_w // chunk
    assert per_w % chunk == 0 and n % nw == 0 and chunk % 8 == 0
    assert n_chunks > _NBUF
    main_chunks = (n_chunks - 2) // _NBUF * _NBUF
    tail = list(range(main_chunks, n_chunks))
    slabs = [(s, min(128, chunk - s)) for s in range(0, chunk, 128)]

    mesh = plsc.VectorSubcoreMesh(core_axis_name="c", subcore_axis_name="s")

    @functools.partial(
        pl.kernel,
        mesh=mesh,
        out_type=jax.ShapeDtypeStruct((n, d), jnp.float32),
        scratch_types=[
            pltpu.VMEM((_NBUF * chunk,), jnp.int32),
            pltpu.VMEM((_NBUF, chunk, d), jnp.float32),
            pltpu.SemaphoreType.DMA((_NBUF,)),
            pltpu.SemaphoreType.DMA((_NBUF,)),
            pltpu.SemaphoreType.DMA((_NBUF,)),
        ],
    )
    def gather_kernel(tok_hbm, idx_hbm, out_hbm, idx_v, rows_v,
                      isem, gsem, osem):
        wid = lax.axis_index("s") * nc + lax.axis_index("c")
        base = wid * per_w

        def start_idx(i, b):
            pltpu.async_copy(
                idx_hbm.at[pl.ds(base + i * chunk, chunk)],
                idx_v.at[pl.ds(b * chunk, chunk)], isem.at[b])

        def wait_idx(i, b):
            pltpu.make_async_copy(
                idx_hbm.at[pl.ds(base + i * chunk, chunk)],
                idx_v.at[pl.ds(b * chunk, chunk)], isem.at[b]).wait()

        def start_gather(b):
            for s, w in slabs:
                pltpu.async_copy(
                    tok_hbm.at[idx_v.at[pl.ds(b * chunk + s, w)]],
                    rows_v.at[b, pl.ds(s, w)], gsem.at[b])

        def wait_gather(b):
            for s, w in slabs:
                pltpu.make_async_copy(
                    tok_hbm.at[idx_v.at[pl.ds(b * chunk + s, w)]],
                    rows_v.at[b, pl.ds(s, w)], gsem.at[b]).wait()

        def start_out(i, b):
            pltpu.async_copy(
                rows_v.at[b], out_hbm.at[pl.ds(base + i * chunk, chunk)],
                osem.at[b])

        def wait_out(i, b):
            pltpu.make_async_copy(
                rows_v.at[b], out_hbm.at[pl.ds(base + i * chunk, chunk)],
                osem.at[b]).wait()

        def maybe_when(cond, fn):
            if isinstance(cond, bool):
                if cond:
                    fn()
            else:
                pl.when(cond)(fn)

        def chunk_step(i, b, issue_next=True):
            bn = (b + 1) % _NBUF
            if issue_next:
                wait_idx(i + 1, bn)
                maybe_when(i + 1 >= _NBUF, lambda: wait_out(i + 1 - _NBUF, bn))
                start_gather(bn)
            wait_gather(b)
            start_out(i, b)
            maybe_when(i + _NBUF < n_chunks, lambda: start_idx(i + _NBUF, b))

        for b in range(_NBUF):
            start_idx(b, b)
        wait_idx(0, 0)
        start_gather(0)

        def super_body(g, carry):
            for b in range(_NBUF):
                chunk_step(g * _NBUF + b, b)
            return carry

        lax.fori_loop(0, main_chunks // _NBUF, super_body, 0)
        for i in tail:
            chunk_step(i, i % _NBUF, issue_next=(i + 1 < n_chunks))
        for i in range(n_chunks - _NBUF, n_chunks):
            wait_out(i, i % _NBUF)

    return gather_kernel(tok_table, flat_idx)


def _tc_add_ln_slice(prev, gathered, pos_slice, gamma2d, beta2d,
                     btotal, row_start, eps=1e-5):
    bsl, l, d = gathered.shape
    b = btotal
    bs = 64
    nblk = bsl // bs
    blk0 = row_start // bs
    inv_d = 1.0 / d

    def body(*refs):
        x_ref, pos_ref, g_ref, bt_ref, o_ref = refs[-5:]
        x = x_ref[...] + pos_ref[...][None, :, :]
        x2 = x.reshape(bs * l, d)
        ones = jnp.ones((d, 128), jnp.float32)
        dn = (((1,), (0,)), ((), ()))
        s = lax.dot_general(x2, ones, dn,
                            preferred_element_type=jnp.float32)[:, :1]
        q = lax.dot_general(x2 * x2, ones, dn,
                            preferred_element_type=jnp.float32)[:, :1]
        mean = s.reshape(bs, l, 1) * inv_d
        var = q.reshape(bs, l, 1) * inv_d - mean * mean
        o_ref[...] = (x - mean) * lax.rsqrt(var + eps) * g_ref[...] + bt_ref[...]

    data_specs = [
        pl.BlockSpec((bs, l, d), lambda i: (i, 0, 0)),
        pl.BlockSpec((l, d), lambda i: (0, 0)),
        pl.BlockSpec((1, d), lambda i: (0, 0)),
        pl.BlockSpec((1, d), lambda i: (0, 0)),
    ]
    prev_args = () if prev is None else (prev,)
    prev_specs = ([] if prev is None
                  else [pl.BlockSpec(memory_space=pltpu.MemorySpace.HBM)])
    return pl.pallas_call(
        body,
        grid=(nblk,),
        in_specs=prev_specs + data_specs,
        out_specs=pl.BlockSpec((bs, l, d),
                               lambda i, blk0=blk0: (i + blk0, 0, 0)),
        out_shape=jax.ShapeDtypeStruct((b, l, d), jnp.float32),
        input_output_aliases={} if prev is None else {0: 0},
        compiler_params=pltpu.CompilerParams(
            dimension_semantics=("arbitrary",)),
    )(*prev_args, gathered, pos_slice, gamma2d, beta2d)


def kernel(indices, tok_table, pos_table, gamma, beta):
    b, l = indices.shape
    d = tok_table.shape[1]
    sizes = [512, 512]
    starts = [0] + [sum(sizes[:i + 1]) for i in range(len(sizes) - 1)]
    flat_idx = indices.reshape(b * l).astype(jnp.int32)
    pos_slice = lax.slice(pos_table, (0, 0), (l, d))
    gamma2d = gamma.reshape(1, d)
    beta2d = beta.reshape(1, d)
    gathered = [
        _sc_gather(tok_table,
                   lax.slice(flat_idx, (st * l,), ((st + sz) * l,)),
                   chunk=l)
        for st, sz in zip(starts, sizes)
    ]
    out = None
    for st, sz, g in zip(starts, sizes, gathered):
        out = _tc_add_ln_slice(out, g.reshape(sz, l, d),
                               pos_slice, gamma2d, beta2d, b, st)
    return out
